# baseline (device time: 39348 ns/iter reference)
import jax
import jax.numpy as jnp
from jax import lax
from jax.experimental import pallas as pl
from jax.experimental.pallas import tpu as pltpu

N_DEV = 32
B, SQ, D = 2, 128, 512
HQ, DH = 4, 64
DHH = HQ * DH
R = B * SQ
ROWS = R // N_DEV


def _body(x_ref, wq_ref, k_ref, v_ref, wo_ref, out_ref,
          q_buf, ctx_buf, y_buf, acc_buf, p1_send, p1_recv, p2_send, p2_recv):
    my_i = lax.axis_index("i")

    q_buf[...] = jnp.dot(
        x_ref[...], wq_ref[...], preferred_element_type=jnp.float32
    ).astype(jnp.bfloat16)

    for bh in range(B * HQ):
        b, h = bh // HQ, bh % HQ
        q_bh = q_buf[b * SQ:(b + 1) * SQ, h * DH:(h + 1) * DH]
        scores = lax.dot_general(
            q_bh, k_ref[bh], (((1,), (1,)), ((), ())),
            preferred_element_type=jnp.float32,
        ) * 0.125
        m = jnp.max(scores, axis=1, keepdims=True)
        e = jnp.exp(scores - m)
        w = (e / jnp.sum(e, axis=1, keepdims=True)).astype(jnp.bfloat16)
        ctx_buf[b * SQ:(b + 1) * SQ, h * DH:(h + 1) * DH] = jnp.dot(
            w, v_ref[bh], preferred_element_type=jnp.float32
        ).astype(jnp.bfloat16)

    y_buf[...] = jnp.dot(
        ctx_buf[...], wo_ref[...], preferred_element_type=jnp.float32
    )

    p1 = []
    for d in range(1, N_DEV):
        partner = my_i ^ d
        rdma = pltpu.make_async_remote_copy(
            src_ref=y_buf.at[pl.ds(partner * ROWS, ROWS)],
            dst_ref=acc_buf.at[d],
            send_sem=p1_send.at[d],
            recv_sem=p1_recv.at[d],
            device_id=(partner,),
            device_id_type=pl.DeviceIdType.MESH,
        )
        rdma.start()
        p1.append(rdma)

    acc_buf[0, :, :] = y_buf[pl.ds(my_i * ROWS, ROWS), :]

    for rdma in p1:
        rdma.wait_recv()
    out_ref[pl.ds(my_i * ROWS, ROWS), :] = jnp.sum(acc_buf[...], axis=0)

    p2 = []
    for d in range(1, N_DEV):
        partner = my_i ^ d
        rdma = pltpu.make_async_remote_copy(
            src_ref=out_ref.at[pl.ds(my_i * ROWS, ROWS)],
            dst_ref=out_ref.at[pl.ds(my_i * ROWS, ROWS)],
            send_sem=p2_send.at[d],
            recv_sem=p2_recv.at[d],
            device_id=(partner,),
            device_id_type=pl.DeviceIdType.MESH,
        )
        rdma.start()
        p2.append(rdma)

    for rdma in p2:
        rdma.wait_recv()
    for rdma in p1 + p2:
        rdma.wait_send()


def kernel(x, Wq, K_ext, V_ext, Wo):
    i = lax.axis_index("i")
    bf16 = jnp.bfloat16

    x2 = x.reshape(R, D).astype(bf16)
    wq_my = lax.dynamic_slice(Wq, (0, i * DHH), (D, DHH)).astype(bf16)
    kT = K_ext.transpose(0, 2, 1, 3).reshape(B * HQ, SQ, DH).astype(bf16)
    vT = V_ext.transpose(0, 2, 1, 3).reshape(B * HQ, SQ, DH).astype(bf16)
    wo_my = lax.dynamic_slice(Wo, (i * DHH, 0), (DHH, D)).astype(bf16)

    out = pl.pallas_call(
        _body,
        out_shape=jax.ShapeDtypeStruct((R, D), jnp.float32),
        in_specs=[pl.BlockSpec(memory_space=pltpu.VMEM)] * 5,
        out_specs=pl.BlockSpec(memory_space=pltpu.VMEM),
        scratch_shapes=[
            pltpu.VMEM((R, DHH), jnp.bfloat16),
            pltpu.VMEM((R, DHH), jnp.bfloat16),
            pltpu.VMEM((R, D), jnp.float32),
            pltpu.VMEM((N_DEV, ROWS, D), jnp.float32),
            pltpu.SemaphoreType.DMA((N_DEV,)),
            pltpu.SemaphoreType.DMA((N_DEV,)),
            pltpu.SemaphoreType.DMA((N_DEV,)),
            pltpu.SemaphoreType.DMA((N_DEV,)),
        ],
    )(x2, wq_my, kT, vT, wo_my)
    return out.reshape(B, SQ, D)
